# Initial kernel scaffold; baseline (speedup 1.0000x reference)
import jax
import jax.numpy as jnp
from jax.experimental import pallas as pl


def kernel(x, mu, var):
    raise NotImplementedError("write your pallas kernel here")


# speedup vs baseline: 1.1016x; 1.1016x over previous
import functools
import math

import jax
import jax.numpy as jnp
from jax import lax
from jax.experimental import pallas as pl
from jax.experimental.pallas import tpu as pltpu

_HIGHEST = lax.Precision.HIGHEST


def _dot(a, b):
    return lax.dot_general(a, b, (((1,), (0,)), ((), ())),
                           precision=_HIGHEST, preferred_element_type=jnp.float32)


def _dot_def(a, b):
    return lax.dot_general(a, b, (((1,), (0,)), ((), ())),
                           preferred_element_type=jnp.float32)


def _r16(t):
    return t.astype(jnp.bfloat16).astype(jnp.float32)


def _prep_codebook(var_ref, mu_ref, ginv_ref, gb_ref, gc_ref, D):
    V = [[var_ref[d * D + e, :][None, :] for e in range(D)] for d in range(D)]
    mu = [mu_ref[d, :][None, :] for d in range(D)]

    L = [[None] * D for _ in range(D)]
    r = [None] * D
    logdet = None
    for j in range(D):
        s = V[j][j]
        for m in range(j):
            s = s - L[j][m] * L[j][m]
        ljj = jnp.sqrt(s)
        L[j][j] = ljj
        r[j] = 1.0 / ljj
        ld = jnp.log(ljj)
        logdet = ld if logdet is None else logdet + ld
        for i in range(j + 1, D):
            t = V[i][j]
            for m in range(j):
                t = t - L[i][m] * L[j][m]
            L[i][j] = t * r[j]
    logdet = 2.0 * logdet

    Linv = [[None] * D for _ in range(D)]
    for j in range(D):
        Linv[j][j] = r[j]
        for i in range(j + 1, D):
            t = L[i][j] * Linv[j][j]
            for m in range(j + 1, i):
                t = t + L[i][m] * Linv[m][j]
            Linv[i][j] = -t * r[i]

    inv = [[None] * D for _ in range(D)]
    for d in range(D):
        for e in range(d, D):
            t = None
            for m in range(max(d, e), D):
                p = Linv[m][d] * Linv[m][e]
                t = p if t is None else t + p
            inv[d][e] = t
            inv[e][d] = t

    b = [None] * D
    for d in range(D):
        t = None
        for e in range(D):
            p = _r16(inv[d][e]) * _r16(mu[e])
            t = p if t is None else t + p
        b[d] = t
    qmm = None
    for d in range(D):
        p = mu[d] * b[d]
        qmm = p if qmm is None else qmm + p

    for d in range(D):
        for e in range(D):
            ginv_ref[d * D + e:d * D + e + 1, :] = inv[d][e]
        gb_ref[d:d + 1, :] = b[d]
    gc_ref[0:1, :] = qmm
    gc_ref[1:2, :] = logdet


def _bitonic_argsort(keys, vals, K):
    lane = lax.broadcasted_iota(jnp.int32, keys.shape, 1)
    k = 2
    while k <= K:
        asc = (lane & k) == 0 if k < K else None
        j = k // 2
        while j >= 1:
            low = (lane & j) == 0
            pk = jnp.where(low, jnp.roll(keys, -j, axis=1), jnp.roll(keys, j, axis=1))
            pv = jnp.where(low, jnp.roll(vals, -j, axis=1), jnp.roll(vals, j, axis=1))
            if asc is None:
                take_min = low
            else:
                take_min = jnp.logical_not(jnp.logical_xor(asc, low))
            cond = ((take_min & (pk < keys))
                    | (jnp.logical_not(take_min) & (pk > keys)))
            keys = jnp.where(cond, pk, keys)
            vals = jnp.where(cond, pv, vals)
            j //= 2
        k *= 2
    return keys, vals


def _kernel_body(x_ref, var_ref, mu_ref, idx_ref, loss_ref,
                 ginv_ref, gb_ref, gc_ref, *, D, K):
    pid = pl.program_id(0)

    @pl.when(pid == 0)
    def _():
        _prep_codebook(var_ref, mu_ref, ginv_ref, gb_ref, gc_ref, D)
        loss_ref[0, 0] = jnp.float32(0.0)

    x = x_ref[...]
    R = x.shape[0]

    jj = lax.broadcasted_iota(jnp.int32, (D, D * D), 1)
    dd = lax.broadcasted_iota(jnp.int32, (D, D * D), 0)
    E1 = (jj // D == dd).astype(jnp.float32)
    E2 = (jj % D == dd).astype(jnp.float32)
    AB = _dot(x, E1) * _dot(x, E2)

    score = _dot_def(AB, ginv_ref[...]) - 2.0 * _dot_def(x, gb_ref[...])
    score = score + gc_ref[0, :][None, :]
    score = score + gc_ref[1, :][None, :]

    vals = lax.broadcasted_iota(jnp.int32, (R, K), 1)
    keys, vals = _bitonic_argsort(score, vals, K)

    idx_ref[...] = vals
    loss_ref[0, 0] += jnp.sum(keys[:, 0:1])


@jax.jit
def kernel(x, mu, var):
    N, D = x.shape
    K = mu.shape[0]
    var_t = var.transpose(1, 2, 0).reshape(D * D, K)
    mu_t = mu.T

    R = 256
    while N % R:
        R //= 2
    grid = (N // R,)

    idx, loss_sum = pl.pallas_call(
        functools.partial(_kernel_body, D=D, K=K),
        grid=grid,
        in_specs=[
            pl.BlockSpec((R, D), lambda i: (i, 0)),
            pl.BlockSpec((D * D, K), lambda i: (0, 0)),
            pl.BlockSpec((D, K), lambda i: (0, 0)),
        ],
        out_specs=[
            pl.BlockSpec((R, K), lambda i: (i, 0)),
            pl.BlockSpec((1, 1), lambda i: (0, 0), memory_space=pltpu.SMEM),
        ],
        out_shape=[
            jax.ShapeDtypeStruct((N, K), jnp.int32),
            jax.ShapeDtypeStruct((1, 1), jnp.float32),
        ],
        scratch_shapes=[
            pltpu.VMEM((D * D, K), jnp.float32),
            pltpu.VMEM((D, K), jnp.float32),
            pltpu.VMEM((2, K), jnp.float32),
        ],
    )(x, var_t, mu_t)

    loss = 0.5 * (loss_sum[0, 0] / N + D * math.log(2.0 * math.pi))
    return idx, loss
